# baseline (speedup 1.0000x reference)
import functools
import jax
import jax.numpy as jnp
from jax import lax
from jax.experimental import pallas as pl
from jax.experimental.pallas import tpu as pltpu
from jax.experimental.pallas import tpu_sc as plsc

R = 2048
N = 32768
NW = 32
RPW = R // NW
L = 16
STEPS = N // L


def _row_argmax(buf):
    iota = lax.broadcasted_iota(jnp.int32, (L,), 0)

    def body(j, carry):
        m, idx, cur = carry
        v = buf[pl.ds(j * L, L)]
        pred = v > m
        m = jnp.where(pred, v, m)
        idx = jnp.where(pred, cur, idx)
        return m, idx, cur + L

    m0 = jnp.full((L,), -jnp.inf, jnp.float32)
    m, idx, _ = lax.fori_loop(0, STEPS, body, (m0, iota * 0, iota), unroll=8)
    M = jnp.max(m)
    return jnp.min(jnp.where(m == M, idx, jnp.int32(2**30)))


def _insert(rvec, lane, val):
    iota = lax.broadcasted_iota(jnp.int32, (L,), 0)
    return jnp.where(iota == lane, val, rvec)


def make_sc_argmax():
    mesh = plsc.VectorSubcoreMesh(core_axis_name="c", subcore_axis_name="s")

    @functools.partial(
        pl.kernel,
        mesh=mesh,
        compiler_params=pltpu.CompilerParams(needs_layout_passes=False),
        out_type=jax.ShapeDtypeStruct((R,), jnp.int32),
        scratch_types=[
            pltpu.VMEM((N,), jnp.float32),
            pltpu.VMEM((N,), jnp.float32),
            pltpu.VMEM((RPW,), jnp.int32),
            pltpu.SemaphoreType.DMA,
            pltpu.SemaphoreType.DMA,
        ],
    )
    def sc_argmax(x_hbm, out_hbm, buf_a, buf_b, res, sem_a, sem_b):
        wid = lax.axis_index("s") * 2 + lax.axis_index("c")
        base = wid * RPW

        pltpu.async_copy(x_hbm.at[base], buf_a, sem_a)
        pltpu.async_copy(x_hbm.at[base + 1], buf_b, sem_b)

        def pair(g, rvec):
            r0 = 2 * g
            pltpu.make_async_copy(x_hbm.at[base], buf_a, sem_a).wait()
            i0 = _row_argmax(buf_a)
            pltpu.async_copy(x_hbm.at[base + r0 + 2], buf_a, sem_a)
            rvec = _insert(rvec, r0 & (L - 1), i0)
            pltpu.make_async_copy(x_hbm.at[base], buf_b, sem_b).wait()
            i1 = _row_argmax(buf_b)
            pltpu.async_copy(x_hbm.at[base + r0 + 3], buf_b, sem_b)
            rvec = _insert(rvec, (r0 + 1) & (L - 1), i1)

            @pl.when((g & 7) == 7)
            def _flush():
                res[pl.ds((g // 8) * L, L)] = rvec

            return rvec

        rvec = jnp.zeros((L,), jnp.int32)
        rvec = lax.fori_loop(0, RPW // 2 - 1, pair, rvec)
        pltpu.make_async_copy(x_hbm.at[base], buf_a, sem_a).wait()
        rvec = _insert(rvec, (RPW - 2) & (L - 1), _row_argmax(buf_a))
        pltpu.make_async_copy(x_hbm.at[base], buf_b, sem_b).wait()
        rvec = _insert(rvec, (RPW - 1) & (L - 1), _row_argmax(buf_b))
        res[pl.ds(RPW - L, L)] = rvec

        pltpu.sync_copy(res, out_hbm.at[pl.ds(base, RPW)])

    return sc_argmax


def kernel(x):
    B, H, n = x.shape
    flat = x.reshape(R, N)
    out = make_sc_argmax()(flat)
    return out.reshape(B, H).astype(jnp.int64)
